# baseline (speedup 1.0000x reference)
import functools

import jax
import jax.numpy as jnp
from jax import lax
from jax.experimental import pallas as pl
from jax.experimental.pallas import tpu as pltpu
from jax.experimental.pallas import tpu_sc as plsc

N_CODES = 8192
DIM = 32
N_TOKENS = 16384


TM = 2048
NCHUNK = 1024
LANES = 128


def _argmin_body(x_ref, e_ref, idx_ref, en_scr, lanes_scr):
    i = pl.program_id(0)

    @pl.when(i == 0)
    def _():
        lanes_scr[...] = lax.broadcasted_iota(
            jnp.int32, (TM, LANES), 1).astype(jnp.float32)
        for c in range(N_CODES // NCHUNK):
            eblk = e_ref[pl.ds(c * NCHUNK, NCHUNK), :]
            en = jnp.sum(eblk * eblk, axis=1)
            en_scr[pl.ds(0, 1), pl.ds(c * NCHUNK, NCHUNK)] = en.reshape(1, -1)

    x = x_ref[...]
    xs = x * (-2.0)

    acc_v = jnp.full((TM, LANES), jnp.inf, dtype=jnp.float32)
    acc_b = jnp.zeros((TM, LANES), dtype=jnp.float32)
    for c in range(N_CODES // NCHUNK):
        eblk = e_ref[pl.ds(c * NCHUNK, NCHUNK), :]
        en = en_scr[pl.ds(0, 1), pl.ds(c * NCHUNK, NCHUNK)]
        dot = lax.dot_general(xs, eblk, (((1,), (1,)), ((), ())),
                              preferred_element_type=jnp.float32)
        d2 = dot + en
        for g in range(NCHUNK // LANES):
            dsl = d2[:, g * LANES:(g + 1) * LANES]
            upd = dsl < acc_v
            acc_v = jnp.minimum(dsl, acc_v)
            acc_b = jnp.where(upd, float(c * NCHUNK + g * LANES), acc_b)

    idxl = acc_b + lanes_scr[...]
    lmin = jnp.min(acc_v, axis=1)
    lif = jnp.min(jnp.where(acc_v == lmin[:, None], idxl, float(N_CODES)),
                  axis=1)
    idx_ref[...] = lif.astype(jnp.int32).reshape(1, 1, TM)


def _tc_argmin(encoded_flat, embedding):
    grid = N_TOKENS // TM
    out = pl.pallas_call(
        _argmin_body,
        grid=(grid,),
        in_specs=[
            pl.BlockSpec((TM, DIM), lambda i: (i, 0)),
            pl.BlockSpec((N_CODES, DIM), lambda i: (0, 0)),
        ],
        out_specs=pl.BlockSpec((1, 1, TM), lambda i: (i, 0, 0)),
        out_shape=jax.ShapeDtypeStruct((grid, 1, TM), jnp.int32),
        scratch_shapes=[pltpu.VMEM((1, N_CODES), jnp.float32),
                        pltpu.VMEM((TM, LANES), jnp.float32)],
    )(encoded_flat, embedding)
    return out.reshape(N_TOKENS)


_SC_NC, _SC_NS = 2, 16
_NW = _SC_NC * _SC_NS
_CH = 128
_BPW = N_TOKENS // _NW
_NCH = _BPW // _CH


@functools.cache
def _sc_gather_fn():
    @functools.partial(
        pl.kernel,
        out_type=jax.ShapeDtypeStruct((N_TOKENS // _CH, _CH, DIM), jnp.float32),
        mesh=plsc.VectorSubcoreMesh(core_axis_name="c", subcore_axis_name="s"),
        scratch_types=[
            pltpu.VMEM((_NCH, _CH), jnp.int32),
            pltpu.VMEM((_NCH, _CH, DIM), jnp.float32),
            pltpu.SemaphoreType.DMA,
        ],
        compiler_params=pltpu.CompilerParams(use_tc_tiling_on_sc=False),
    )
    def _sc_gather(table_hbm, idx_hbm, out_hbm, idx_v, rows_v, sem):
        wid = lax.axis_index("s") * _SC_NC + lax.axis_index("c")
        base = wid * _NCH
        pltpu.sync_copy(idx_hbm.at[pl.ds(base, _NCH)], idx_v)
        copies = [
            pltpu.async_copy(table_hbm.at[idx_v.at[j]], rows_v.at[j], sem)
            for j in range(_NCH)
        ]
        for cp in copies:
            cp.wait()
        pltpu.sync_copy(rows_v, out_hbm.at[pl.ds(base, _NCH)])

    return _sc_gather


def kernel(z, embedding):
    encoded_permuted = jnp.transpose(z, (0, 2, 3, 1))
    permuted_shape = encoded_permuted.shape
    encoded_flat = encoded_permuted.reshape(-1, DIM)

    codebook_indices = _tc_argmin(encoded_flat, embedding)

    rows = _sc_gather_fn()(embedding, codebook_indices.reshape(N_TOKENS // _CH, _CH))
    quantized_flat = rows.reshape(N_TOKENS, DIM)

    quantized = jnp.transpose(quantized_flat.reshape(permuted_shape), (0, 3, 1, 2))
    return (encoded_flat, quantized_flat, codebook_indices, quantized)
